# Initial kernel scaffold; baseline (speedup 1.0000x reference)
import jax
import jax.numpy as jnp
from jax.experimental import pallas as pl


def kernel(query, database, database_labels):
    raise NotImplementedError("write your pallas kernel here")


# speedup vs baseline: 1.5336x; 1.5336x over previous
import functools

import jax
import jax.numpy as jnp
from jax import lax
from jax.experimental import pallas as pl
from jax.experimental.pallas import tpu as pltpu
from jax.experimental.pallas import tpu_sc as plsc

Q = 1024
D = 128
N = 100000
K = 8
OUT_DIM = 32

BN = 2048
NT = (N + BN - 1) // BN
N_PAD = NT * BN

_BIG_I = 2**30


def _topk_tile_kernel(q_ref, db_ref, vals_ref, idxs_ref):
    q = q_ref[...]
    db = db_ref[...]
    qd = lax.dot_general(q, db, (((1,), (1,)), ((), ())),
                         preferred_element_type=jnp.float32)
    q_sq = jnp.sum(q * q, axis=1, keepdims=True)
    d_sq = jnp.sum(db * db, axis=1)[None, :]
    d2 = q_sq + d_sq - 2.0 * qd
    dist = jnp.sqrt(jnp.maximum(d2, 0.0))

    base = pl.program_id(0) * BN
    col = lax.broadcasted_iota(jnp.int32, (Q, BN), 1)
    dist = jnp.where(col + base >= N, jnp.inf, dist)

    for k in range(K):
        m = jnp.min(dist, axis=1)
        cidx = jnp.min(jnp.where(dist == m[:, None], col, _BIG_I),
                       axis=1)
        vals_ref[0, k, :] = m
        idxs_ref[0, k, :] = cidx + base
        dist = jnp.where(col == cidx[:, None], jnp.inf, dist)


def _merge_kernel(vals_ref, idxs_ref, out_ref):
    v = vals_ref[...]
    x = idxs_ref[...]
    for k in range(K):
        m = jnp.min(v, axis=0)
        gi = jnp.min(jnp.where(v == m[None, :], x, _BIG_I), axis=0)
        out_ref[k, :] = gi * 2 + (k // 4)
        v = jnp.where(x == gi[None, :], jnp.inf, v)


def _make_sc_gather_mean():
    info = plsc.get_sparse_core_info()
    nc, ns = info.num_cores, info.num_subcores
    nw = nc * ns
    b_per_w = (Q * K) // nw
    q_per_w = Q // nw
    n_chunk = b_per_w // 128
    mesh = plsc.VectorSubcoreMesh(core_axis_name="c", subcore_axis_name="s")

    @functools.partial(
        pl.kernel, mesh=mesh,
        out_type=jax.ShapeDtypeStruct((Q, OUT_DIM), jnp.float32),
        scratch_types=[
            pltpu.VMEM((n_chunk, 128), jnp.int32),
            pltpu.VMEM((b_per_w, 128), jnp.float32),
            pltpu.VMEM((q_per_w, OUT_DIM), jnp.float32),
            pltpu.SemaphoreType.DMA,
        ],
    )
    def sc_gather_mean(labels_hbm, fidx_hbm, out_hbm, idx_v, rows_v, out_v,
                       sem):
        wid = lax.axis_index("s") * nc + lax.axis_index("c")
        pltpu.sync_copy(fidx_hbm.at[wid], idx_v)
        copies = [
            pltpu.async_copy(labels_hbm.at[idx_v.at[b]],
                             rows_v.at[pl.ds(b * 128, 128)], sem)
            for b in range(n_chunk)
        ]
        for c in copies:
            c.wait()

        def body(r, carry):
            for h in range(OUT_DIM // 16):
                acc = rows_v[r * K, pl.ds(h * 16, 16)]
                for kk in range(1, K):
                    off = (kk % 4) * OUT_DIM + h * 16
                    acc = acc + rows_v[r * K + kk, pl.ds(off, 16)]
                out_v[r, pl.ds(h * 16, 16)] = acc * (1.0 / K)
            return carry

        lax.fori_loop(0, q_per_w, body, 0)
        pltpu.sync_copy(out_v, out_hbm.at[pl.ds(wid * q_per_w, q_per_w)])

    return sc_gather_mean


_sc_cache = []


def _get_sc_gather_mean():
    if not _sc_cache:
        _sc_cache.append(_make_sc_gather_mean())
    return _sc_cache[0]


def kernel(query, database, database_labels):
    db_pad = jnp.pad(database, ((0, N_PAD - N), (0, 0)))

    vals, idxs = pl.pallas_call(
        _topk_tile_kernel,
        grid=(NT,),
        in_specs=[
            pl.BlockSpec((Q, D), lambda i: (0, 0)),
            pl.BlockSpec((BN, D), lambda i: (i, 0)),
        ],
        out_specs=[
            pl.BlockSpec((1, K, Q), lambda i: (i, 0, 0)),
            pl.BlockSpec((1, K, Q), lambda i: (i, 0, 0)),
        ],
        out_shape=[
            jax.ShapeDtypeStruct((NT, K, Q), jnp.float32),
            jax.ShapeDtypeStruct((NT, K, Q), jnp.int32),
        ],
    )(query, db_pad)

    fidx = pl.pallas_call(
        _merge_kernel,
        out_shape=jax.ShapeDtypeStruct((K, Q), jnp.int32),
    )(vals.reshape(NT * K, Q), idxs.reshape(NT * K, Q))

    labels_flat = database_labels.reshape(N * 2, 128)
    fidx_flat = fidx.T.reshape(32, 2, 128)
    return _get_sc_gather_mean()(labels_flat, fidx_flat)


# speedup vs baseline: 1.5543x; 1.0135x over previous
import functools

import jax
import jax.numpy as jnp
from jax import lax
from jax.experimental import pallas as pl
from jax.experimental.pallas import tpu as pltpu
from jax.experimental.pallas import tpu_sc as plsc

Q = 1024
D = 128
N = 100000
K = 8
OUT_DIM = 32

BN = 2000
NT = N // BN

_BIG_I = 2**30


def _topk_tile_kernel(q_ref, db_ref, vals_ref, idxs_ref):
    q = q_ref[...]
    db = db_ref[...]
    qd = lax.dot_general(q, db, (((1,), (1,)), ((), ())),
                         preferred_element_type=jnp.float32)
    q_sq = jnp.sum(q * q, axis=1, keepdims=True)
    d_sq = jnp.sum(db * db, axis=1)[None, :]
    d2 = q_sq + d_sq - 2.0 * qd
    dist = jnp.sqrt(jnp.maximum(d2, 0.0))

    base = pl.program_id(0) * BN
    col = lax.broadcasted_iota(jnp.int32, (Q, BN), 1)

    for k in range(K):
        m = jnp.min(dist, axis=1)
        cidx = jnp.min(jnp.where(dist == m[:, None], col, _BIG_I),
                       axis=1)
        vals_ref[0, k, :] = m
        idxs_ref[0, k, :] = cidx + base
        dist = jnp.where(col == cidx[:, None], jnp.inf, dist)


def _merge_kernel(vals_ref, idxs_ref, out_ref):
    v = vals_ref[...]
    x = idxs_ref[...]
    for k in range(K):
        m = jnp.min(v, axis=0)
        gi = jnp.min(jnp.where(v == m[None, :], x, _BIG_I), axis=0)
        out_ref[k, :] = gi * 2 + (k // 4)
        v = jnp.where(x == gi[None, :], jnp.inf, v)


def _make_sc_gather_mean():
    info = plsc.get_sparse_core_info()
    nc, ns = info.num_cores, info.num_subcores
    nw = nc * ns
    b_per_w = (Q * K) // nw
    q_per_w = Q // nw
    n_chunk = b_per_w // 128
    mesh = plsc.VectorSubcoreMesh(core_axis_name="c", subcore_axis_name="s")

    @functools.partial(
        pl.kernel, mesh=mesh,
        out_type=jax.ShapeDtypeStruct((Q, OUT_DIM), jnp.float32),
        scratch_types=[
            pltpu.VMEM((n_chunk, 128), jnp.int32),
            pltpu.VMEM((b_per_w, 128), jnp.float32),
            pltpu.VMEM((q_per_w, OUT_DIM), jnp.float32),
            pltpu.SemaphoreType.DMA,
        ],
    )
    def sc_gather_mean(labels_hbm, fidx_hbm, out_hbm, idx_v, rows_v, out_v,
                       sem):
        wid = lax.axis_index("s") * nc + lax.axis_index("c")
        pltpu.sync_copy(fidx_hbm.at[wid], idx_v)
        copies = [
            pltpu.async_copy(labels_hbm.at[idx_v.at[b]],
                             rows_v.at[pl.ds(b * 128, 128)], sem)
            for b in range(n_chunk)
        ]
        for c in copies:
            c.wait()

        def body(r, carry):
            for h in range(OUT_DIM // 16):
                acc = rows_v[r * K, pl.ds(h * 16, 16)]
                for kk in range(1, K):
                    off = (kk % 4) * OUT_DIM + h * 16
                    acc = acc + rows_v[r * K + kk, pl.ds(off, 16)]
                out_v[r, pl.ds(h * 16, 16)] = acc * (1.0 / K)
            return carry

        lax.fori_loop(0, q_per_w, body, 0)
        pltpu.sync_copy(out_v, out_hbm.at[pl.ds(wid * q_per_w, q_per_w)])

    return sc_gather_mean


_sc_cache = []


def _get_sc_gather_mean():
    if not _sc_cache:
        _sc_cache.append(_make_sc_gather_mean())
    return _sc_cache[0]


def kernel(query, database, database_labels):
    vals, idxs = pl.pallas_call(
        _topk_tile_kernel,
        grid=(NT,),
        in_specs=[
            pl.BlockSpec((Q, D), lambda i: (0, 0)),
            pl.BlockSpec((BN, D), lambda i: (i, 0)),
        ],
        out_specs=[
            pl.BlockSpec((1, K, Q), lambda i: (i, 0, 0)),
            pl.BlockSpec((1, K, Q), lambda i: (i, 0, 0)),
        ],
        out_shape=[
            jax.ShapeDtypeStruct((NT, K, Q), jnp.float32),
            jax.ShapeDtypeStruct((NT, K, Q), jnp.int32),
        ],
    )(query, database)

    fidx = pl.pallas_call(
        _merge_kernel,
        out_shape=jax.ShapeDtypeStruct((K, Q), jnp.int32),
    )(vals.reshape(NT * K, Q), idxs.reshape(NT * K, Q))

    labels_flat = database_labels.reshape(N * 2, 128)
    fidx_flat = fidx.T.reshape(32, 2, 128)
    return _get_sc_gather_mean()(labels_flat, fidx_flat)
